# Initial kernel scaffold; baseline (speedup 1.0000x reference)
import jax
import jax.numpy as jnp
from jax.experimental import pallas as pl


def kernel(values, lengths):
    raise NotImplementedError("write your pallas kernel here")


# speedup vs baseline: 3494.5931x; 3494.5931x over previous
import functools

import numpy as np
import jax
import jax.numpy as jnp
from jax import lax
from jax.experimental import pallas as pl
from jax.experimental.pallas import tpu as pltpu
from jax.experimental.pallas import tpu_sc as plsc

W = 8
LOCAL_SPLIT = 13
B = 1024
AVG_LEN = 20
NBLOCKS = W * LOCAL_SPLIT
TOTAL = NBLOCKS * B * AVG_LEN
NPAD = 112

_info = plsc.get_sparse_core_info()
NC, NS, LN = _info.num_cores, _info.num_subcores, _info.num_lanes
NW = NC * NS
BLOCKS_PER_SUB = -(-NBLOCKS // NS)
BLOCKS_PER_WORKER = -(-NBLOCKS // NW)

C = 8192
NCHUNK = TOTAL // C
CH_PER_W = -(-NCHUNK // NW)
NPAIR = (CH_PER_W + 1) // 2
R = 1024
NSTG = (C + 7 + R - 1) // R
STAGE = NSTG * R


def _recat_perm() -> np.ndarray:
    out = []
    for i in range(LOCAL_SPLIT):
        for j in range(W):
            out.append(i + j * LOCAL_SPLIT)
    return np.array(out, dtype=np.int32)


def _scalar_at(ref, i):
    return plsc.load_gather(ref, [jnp.full((LN,), i, jnp.int32)])[0]


def _al(x, n=8):
    return pl.multiple_of(x, n)


def _body(vals, lens, recat_h, vout, lout,
          recat_v, lenbuf, srow_v, sums_v,
          bs_v, inoff_v, src_v, dst_v, asm0, asm1, stage, spm_sums,
          rsem, wsem):
    c = lax.axis_index("c")
    s = lax.axis_index("s")
    wid = s * NC + c
    iota = lax.iota(jnp.int32, LN)

    pltpu.sync_copy(recat_h, recat_v)

    srow_v[...] = jnp.zeros((LN,), jnp.int32)
    for t in range(BLOCKS_PER_SUB):
        jc = jnp.minimum(s + NS * t, NBLOCKS - 1)
        pltpu.async_copy(lens.at[pl.ds(_al(jc * B), B)],
                         lenbuf.at[pl.ds(t * B, B)], rsem)
    for t in range(BLOCKS_PER_SUB):
        jc = jnp.minimum(s + NS * t, NBLOCKS - 1)
        pltpu.make_async_copy(lens.at[pl.ds(_al(jc * B), B)],
                              lenbuf.at[pl.ds(t * B, B)], rsem).wait()
    for t in range(BLOCKS_PER_SUB):
        j = s + NS * t
        acc = jnp.zeros((LN,), jnp.int32)
        for q in range(B // LN):
            acc = acc + lenbuf[pl.ds(t * B + q * LN, LN)]
        for kk in (1, 2, 4, 8):
            acc = acc + jnp.take(acc, jnp.bitwise_xor(iota, kk))
        ssum = jnp.where(j < NBLOCKS, acc[0], 0)
        plsc.store_scatter(srow_v, [jnp.full((LN,), t, jnp.int32)],
                           jnp.full((LN,), ssum, jnp.int32))
    pltpu.sync_copy(srow_v, spm_sums.at[pl.ds(_al(s * LN), LN)])
    plsc.subcore_barrier()
    pltpu.sync_copy(spm_sums, sums_v)

    cin = jnp.int32(0)
    for v in range(NPAD // LN):
        j = v * LN + iota
        idx = (j % NS) * LN + (j // NS)
        bsv = plsc.load_gather(sums_v, [idx])
        bsv = jnp.where(j < NBLOCKS, bsv, 0)
        incl = plsc.cumsum(bsv)
        bs_v[pl.ds(v * LN, LN)] = bsv
        inoff_v[pl.ds(v * LN, LN)] = incl - bsv + cin
        cin = cin + jnp.sum(bsv)

    cout = jnp.int32(0)
    for v in range(NPAD // LN):
        i = v * LN + iota
        ic = jnp.minimum(i, NBLOCKS - 1)
        rc = plsc.load_gather(recat_v, [ic])
        pbs = plsc.load_gather(bs_v, [rc])
        pbs = jnp.where(i < NBLOCKS, pbs, 0)
        sb = plsc.load_gather(inoff_v, [rc])
        incl = plsc.cumsum(pbs)
        src_v[pl.ds(v * LN, LN)] = sb
        dst_v[pl.ds(v * LN, LN)] = incl - pbs + cout
        cout = cout + jnp.sum(pbs)

    for t in range(BLOCKS_PER_WORKER):
        i = wid + NW * t
        ic = jnp.minimum(i, NBLOCKS - 1)
        r = _scalar_at(recat_v, ic)

        @pl.when(i < NBLOCKS)
        def _row(r=r, ic=ic):
            pltpu.sync_copy(lens.at[pl.ds(_al(r * B), B)],
                            lenbuf.at[pl.ds(0, B)])
            pltpu.sync_copy(lenbuf.at[pl.ds(0, B)],
                            lout.at[pl.ds(_al(ic * B), B)])

    def assemble(g, asmb):
        o0 = _al(g * C)
        acc_le = jnp.zeros((LN,), jnp.int32)
        acc_lt = jnp.zeros((LN,), jnp.int32)
        for v in range(NPAD // LN):
            dv = dst_v[pl.ds(v * LN, LN)]
            acc_le = acc_le + (dv <= o0).astype(jnp.int32)
            acc_lt = acc_lt + (dv < o0 + C).astype(jnp.int32)
        for kk in (1, 2, 4, 8):
            acc_le = acc_le + jnp.take(acc_le, jnp.bitwise_xor(iota, kk))
            acc_lt = acc_lt + jnp.take(acc_lt, jnp.bitwise_xor(iota, kk))
        jlo = acc_le[0] - 1
        jhi = acc_lt[0]

        def seg_body(j, _):
            dj = _scalar_at(dst_v, j)
            dj1 = _scalar_at(dst_v, j + 1)
            a = jnp.maximum(dj - o0, 0)
            b = jnp.minimum(dj1 - o0, C)

            @pl.when(b > a)
            def _seg(j=j, a=a, b=b, dj=dj):
                sj = _scalar_at(src_v, j)
                sA = sj + (o0 + a - dj)
                sh = jnp.bitwise_and(sA, 7)
                sA8 = sA - sh
                s0 = jnp.minimum(sA8, TOTAL - R)
                e = (sA8 - s0) + sh + (b - a)
                nr = (e + R - 1) // R

                def fire(q, _):
                    so = jnp.minimum(s0 + q * R, TOTAL - R)
                    pltpu.async_copy(
                        vals.at[pl.ds(_al(so), R)],
                        stage.at[pl.ds(_al(so - s0), R)], rsem)
                    return 0

                def drain(q, _):
                    so = jnp.minimum(s0 + q * R, TOTAL - R)
                    pltpu.make_async_copy(
                        vals.at[pl.ds(_al(so), R)],
                        stage.at[pl.ds(_al(so - s0), R)], rsem).wait()
                    return 0

                lax.fori_loop(0, nr, fire, 0)
                lax.fori_loop(0, nr, drain, 0)

                d = (sA - s0) - a
                va = a // LN
                vb = (b - 1) // LN

                def edge(v):
                    base = _al(v * LN, LN)
                    x = base + iota
                    m = (x >= a) & (x < b)
                    gi = jnp.clip(x + d, 0, STAGE - 1)
                    gv = plsc.load_gather(stage, [gi])
                    old = asmb[pl.ds(base, LN)]
                    asmb[pl.ds(base, LN)] = jnp.where(m, gv, old)

                edge(va)

                @pl.when(vb > va)
                def _hi():
                    edge(vb)

                def interior(v, _):
                    base = _al(v * LN, LN)
                    gv = plsc.load_gather(stage, [base + iota + d])
                    asmb[pl.ds(base, LN)] = gv
                    return 0

                lax.fori_loop(va + 1, vb, interior, 0)

            return 0

        lax.fori_loop(jlo, jhi, seg_body, 0)

    def chunk_pair(tt, _):
        g0 = wid + NW * (2 * tt)
        g1 = g0 + NW

        @pl.when(g0 < NCHUNK)
        def _c0(g0=g0):
            assemble(g0, asm0)
            pltpu.async_copy(asm0, vout.at[pl.ds(_al(g0 * C), C)], wsem)

        @pl.when(g1 < NCHUNK)
        def _c1(g1=g1):
            assemble(g1, asm1)
            pltpu.async_copy(asm1, vout.at[pl.ds(_al(g1 * C), C)], wsem)

        @pl.when(g0 < NCHUNK)
        def _w0(g0=g0):
            pltpu.make_async_copy(asm0, vout.at[pl.ds(_al(g0 * C), C)],
                                  wsem).wait()

        @pl.when(g1 < NCHUNK)
        def _w1(g1=g1):
            pltpu.make_async_copy(asm1, vout.at[pl.ds(_al(g1 * C), C)],
                                  wsem).wait()

        return 0

    lax.fori_loop(0, NPAIR, chunk_pair, 0)


@functools.partial(
    pl.kernel,
    out_type=[
        jax.ShapeDtypeStruct((TOTAL,), jnp.float32),
        jax.ShapeDtypeStruct((NBLOCKS * B,), jnp.int32),
    ],
    mesh=plsc.VectorSubcoreMesh(core_axis_name="c", subcore_axis_name="s"),
    scratch_types=[
        pltpu.VMEM((NPAD,), jnp.int32),
        pltpu.VMEM((BLOCKS_PER_SUB * B,), jnp.int32),
        pltpu.VMEM((LN,), jnp.int32),
        pltpu.VMEM((NS * LN,), jnp.int32),
        pltpu.VMEM((NPAD,), jnp.int32),
        pltpu.VMEM((NPAD,), jnp.int32),
        pltpu.VMEM((NPAD,), jnp.int32),
        pltpu.VMEM((NPAD,), jnp.int32),
        pltpu.VMEM((C,), jnp.float32),
        pltpu.VMEM((C,), jnp.float32),
        pltpu.VMEM((STAGE,), jnp.float32),
        pltpu.VMEM_SHARED((NS * LN,), jnp.int32),
        pltpu.SemaphoreType.DMA,
        pltpu.SemaphoreType.DMA,
    ],
    compiler_params=pltpu.CompilerParams(needs_layout_passes=False),
)
def _kjt_recat(vals, lens, recat_h, vout, lout, *scratch):
    _body(vals, lens, recat_h, vout, lout, *scratch)


def kernel(values, lengths):
    recat = jnp.asarray(np.pad(_recat_perm(), (0, NPAD - NBLOCKS)))
    values_out, lengths_out = _kjt_recat(values, lengths, recat)
    return values_out, lengths_out


# speedup vs baseline: 4475.1183x; 1.2806x over previous
import functools

import numpy as np
import jax
import jax.numpy as jnp
from jax import lax
from jax.experimental import pallas as pl
from jax.experimental.pallas import tpu as pltpu
from jax.experimental.pallas import tpu_sc as plsc

W = 8
LOCAL_SPLIT = 13
B = 1024
AVG_LEN = 20
NBLOCKS = W * LOCAL_SPLIT
TOTAL = NBLOCKS * B * AVG_LEN
NPAD = 112

_info = plsc.get_sparse_core_info()
NC, NS, LN = _info.num_cores, _info.num_subcores, _info.num_lanes
NW = NC * NS
BLOCKS_PER_SUB = -(-NBLOCKS // NS)
BLOCKS_PER_WORKER = -(-NBLOCKS // NW)

C = 8192
NCHUNK = TOTAL // C
CH_PER_W = -(-NCHUNK // NW)
NPAIR = (CH_PER_W + 1) // 2
R = 1024
NSTG = (C + 7 + R - 1) // R
STAGE = NSTG * R


def _recat_perm() -> np.ndarray:
    out = []
    for i in range(LOCAL_SPLIT):
        for j in range(W):
            out.append(i + j * LOCAL_SPLIT)
    return np.array(out, dtype=np.int32)


def _scalar_at(ref, i):
    return plsc.load_gather(ref, [jnp.full((LN,), i, jnp.int32)])[0]


def _al(x, n=8):
    return pl.multiple_of(x, n)


def _body(vals, lens, recat_h, vout, lout,
          recat_v, lenbuf, srow_v, sums_v,
          bs_v, inoff_v, src_v, dst_v, asm0, asm1, stage, spm_sums,
          rsem, wsem):
    c = lax.axis_index("c")
    s = lax.axis_index("s")
    wid = s * NC + c
    iota = lax.iota(jnp.int32, LN)

    pltpu.sync_copy(recat_h, recat_v)

    srow_v[...] = jnp.zeros((LN,), jnp.int32)
    for t in range(BLOCKS_PER_SUB):
        jc = jnp.minimum(s + NS * t, NBLOCKS - 1)
        pltpu.async_copy(lens.at[pl.ds(_al(jc * B), B)],
                         lenbuf.at[pl.ds(t * B, B)], rsem)
    for t in range(BLOCKS_PER_SUB):
        jc = jnp.minimum(s + NS * t, NBLOCKS - 1)
        pltpu.make_async_copy(lens.at[pl.ds(_al(jc * B), B)],
                              lenbuf.at[pl.ds(t * B, B)], rsem).wait()
    for t in range(BLOCKS_PER_SUB):
        j = s + NS * t
        acc = jnp.zeros((LN,), jnp.int32)
        for q in range(B // LN):
            acc = acc + lenbuf[pl.ds(t * B + q * LN, LN)]
        for kk in (1, 2, 4, 8):
            acc = acc + jnp.take(acc, jnp.bitwise_xor(iota, kk))
        ssum = jnp.where(j < NBLOCKS, acc[0], 0)
        plsc.store_scatter(srow_v, [jnp.full((LN,), t, jnp.int32)],
                           jnp.full((LN,), ssum, jnp.int32))
    pltpu.sync_copy(srow_v, spm_sums.at[pl.ds(_al(s * LN), LN)])
    plsc.subcore_barrier()
    pltpu.sync_copy(spm_sums, sums_v)

    cin = jnp.int32(0)
    for v in range(NPAD // LN):
        j = v * LN + iota
        idx = (j % NS) * LN + (j // NS)
        bsv = plsc.load_gather(sums_v, [idx])
        bsv = jnp.where(j < NBLOCKS, bsv, 0)
        incl = plsc.cumsum(bsv)
        bs_v[pl.ds(v * LN, LN)] = bsv
        inoff_v[pl.ds(v * LN, LN)] = incl - bsv + cin
        cin = cin + jnp.sum(bsv)

    cout = jnp.int32(0)
    for v in range(NPAD // LN):
        i = v * LN + iota
        ic = jnp.minimum(i, NBLOCKS - 1)
        rc = plsc.load_gather(recat_v, [ic])
        pbs = plsc.load_gather(bs_v, [rc])
        pbs = jnp.where(i < NBLOCKS, pbs, 0)
        sb = plsc.load_gather(inoff_v, [rc])
        incl = plsc.cumsum(pbs)
        src_v[pl.ds(v * LN, LN)] = sb
        dst_v[pl.ds(v * LN, LN)] = incl - pbs + cout
        cout = cout + jnp.sum(pbs)

    for t in range(BLOCKS_PER_WORKER):
        i = wid + NW * t
        ic = jnp.minimum(i, NBLOCKS - 1)
        r = _scalar_at(recat_v, ic)

        @pl.when(i < NBLOCKS)
        def _row(r=r, ic=ic):
            pltpu.sync_copy(lens.at[pl.ds(_al(r * B), B)],
                            lenbuf.at[pl.ds(0, B)])
            pltpu.sync_copy(lenbuf.at[pl.ds(0, B)],
                            lout.at[pl.ds(_al(ic * B), B)])

    def assemble(g, asmb):
        o0 = _al(g * C)
        acc_le = jnp.zeros((LN,), jnp.int32)
        acc_lt = jnp.zeros((LN,), jnp.int32)
        for v in range(NPAD // LN):
            dv = dst_v[pl.ds(v * LN, LN)]
            acc_le = acc_le + (dv <= o0).astype(jnp.int32)
            acc_lt = acc_lt + (dv < o0 + C).astype(jnp.int32)
        for kk in (1, 2, 4, 8):
            acc_le = acc_le + jnp.take(acc_le, jnp.bitwise_xor(iota, kk))
            acc_lt = acc_lt + jnp.take(acc_lt, jnp.bitwise_xor(iota, kk))
        jlo = acc_le[0] - 1
        jhi = acc_lt[0]

        def seg_body(j, _):
            dj = _scalar_at(dst_v, j)
            dj1 = _scalar_at(dst_v, j + 1)
            a = jnp.maximum(dj - o0, 0)
            b = jnp.minimum(dj1 - o0, C)

            @pl.when(b > a)
            def _seg(j=j, a=a, b=b, dj=dj):
                sj = _scalar_at(src_v, j)
                sA = sj + (o0 + a - dj)
                sh = jnp.bitwise_and(sA, 7)
                sA8 = sA - sh
                s0 = jnp.minimum(sA8, TOTAL - R)
                e = (sA8 - s0) + sh + (b - a)
                nr = (e + R - 1) // R

                def fire(q, _):
                    so = jnp.minimum(s0 + q * R, TOTAL - R)
                    pltpu.async_copy(
                        vals.at[pl.ds(_al(so), R)],
                        stage.at[pl.ds(_al(so - s0), R)], rsem)
                    return 0

                def drain(q, _):
                    so = jnp.minimum(s0 + q * R, TOTAL - R)
                    pltpu.make_async_copy(
                        vals.at[pl.ds(_al(so), R)],
                        stage.at[pl.ds(_al(so - s0), R)], rsem).wait()
                    return 0

                lax.fori_loop(0, nr, fire, 0)
                lax.fori_loop(0, nr, drain, 0)

                d = (sA - s0) - a
                va = a // LN
                vb = (b - 1) // LN

                def edge(v):
                    base = _al(v * LN, LN)
                    x = base + iota
                    m = (x >= a) & (x < b)
                    gi = jnp.clip(x + d, 0, STAGE - 1)
                    gv = plsc.load_gather(stage, [gi])
                    old = asmb[pl.ds(base, LN)]
                    asmb[pl.ds(base, LN)] = jnp.where(m, gv, old)

                edge(va)

                @pl.when(vb > va)
                def _hi():
                    edge(vb)

                @plsc.parallel_loop(va + 1, vb, unroll=8)
                def _interior(v):
                    base = _al(v * LN, LN)
                    gv = plsc.load_gather(stage, [base + iota + d])
                    asmb[pl.ds(base, LN)] = gv

            return 0

        lax.fori_loop(jlo, jhi, seg_body, 0)

    def chunk_pair(tt, _):
        g0 = wid + NW * (2 * tt)
        g1 = g0 + NW

        @pl.when(g0 < NCHUNK)
        def _c0(g0=g0):
            assemble(g0, asm0)
            pltpu.async_copy(asm0, vout.at[pl.ds(_al(g0 * C), C)], wsem)

        @pl.when(g1 < NCHUNK)
        def _c1(g1=g1):
            assemble(g1, asm1)
            pltpu.async_copy(asm1, vout.at[pl.ds(_al(g1 * C), C)], wsem)

        @pl.when(g0 < NCHUNK)
        def _w0(g0=g0):
            pltpu.make_async_copy(asm0, vout.at[pl.ds(_al(g0 * C), C)],
                                  wsem).wait()

        @pl.when(g1 < NCHUNK)
        def _w1(g1=g1):
            pltpu.make_async_copy(asm1, vout.at[pl.ds(_al(g1 * C), C)],
                                  wsem).wait()

        return 0

    lax.fori_loop(0, NPAIR, chunk_pair, 0)


@functools.partial(
    pl.kernel,
    out_type=[
        jax.ShapeDtypeStruct((TOTAL,), jnp.float32),
        jax.ShapeDtypeStruct((NBLOCKS * B,), jnp.int32),
    ],
    mesh=plsc.VectorSubcoreMesh(core_axis_name="c", subcore_axis_name="s"),
    scratch_types=[
        pltpu.VMEM((NPAD,), jnp.int32),
        pltpu.VMEM((BLOCKS_PER_SUB * B,), jnp.int32),
        pltpu.VMEM((LN,), jnp.int32),
        pltpu.VMEM((NS * LN,), jnp.int32),
        pltpu.VMEM((NPAD,), jnp.int32),
        pltpu.VMEM((NPAD,), jnp.int32),
        pltpu.VMEM((NPAD,), jnp.int32),
        pltpu.VMEM((NPAD,), jnp.int32),
        pltpu.VMEM((C,), jnp.float32),
        pltpu.VMEM((C,), jnp.float32),
        pltpu.VMEM((STAGE,), jnp.float32),
        pltpu.VMEM_SHARED((NS * LN,), jnp.int32),
        pltpu.SemaphoreType.DMA,
        pltpu.SemaphoreType.DMA,
    ],
    compiler_params=pltpu.CompilerParams(needs_layout_passes=False),
)
def _kjt_recat(vals, lens, recat_h, vout, lout, *scratch):
    _body(vals, lens, recat_h, vout, lout, *scratch)


def kernel(values, lengths):
    recat = jnp.asarray(np.pad(_recat_perm(), (0, NPAD - NBLOCKS)))
    values_out, lengths_out = _kjt_recat(values, lengths, recat)
    return values_out, lengths_out


# speedup vs baseline: 4627.0913x; 1.0340x over previous
import functools

import numpy as np
import jax
import jax.numpy as jnp
from jax import lax
from jax.experimental import pallas as pl
from jax.experimental.pallas import tpu as pltpu
from jax.experimental.pallas import tpu_sc as plsc

W = 8
LOCAL_SPLIT = 13
B = 1024
AVG_LEN = 20
NBLOCKS = W * LOCAL_SPLIT
TOTAL = NBLOCKS * B * AVG_LEN
NPAD = 112

_info = plsc.get_sparse_core_info()
NC, NS, LN = _info.num_cores, _info.num_subcores, _info.num_lanes
NW = NC * NS
BLOCKS_PER_SUB = -(-NBLOCKS // NS)
BLOCKS_PER_WORKER = -(-NBLOCKS // NW)

C = 8192
NCHUNK = TOTAL // C
CH_PER_W = -(-NCHUNK // NW)
NPAIR = (CH_PER_W + 1) // 2
R = 1024
NSTG = (C + 7 + R - 1) // R
STAGE = NSTG * R


def _recat_perm() -> np.ndarray:
    out = []
    for i in range(LOCAL_SPLIT):
        for j in range(W):
            out.append(i + j * LOCAL_SPLIT)
    return np.array(out, dtype=np.int32)


def _scalar_at(ref, i):
    return plsc.load_gather(ref, [jnp.full((LN,), i, jnp.int32)])[0]


def _al(x, n=8):
    return pl.multiple_of(x, n)


def _body(vals, lens, recat_h, vout, lout,
          recat_v, lenbuf, srow_v, sums_v,
          bs_v, inoff_v, src_v, dst_v, asm0, asm1, stage, spm_sums,
          rsem, wsem, lsem):
    c = lax.axis_index("c")
    s = lax.axis_index("s")
    wid = s * NC + c
    iota = lax.iota(jnp.int32, LN)

    pltpu.sync_copy(recat_h, recat_v)

    srow_v[...] = jnp.zeros((LN,), jnp.int32)
    for t in range(BLOCKS_PER_SUB):
        jc = jnp.minimum(s + NS * t, NBLOCKS - 1)
        pltpu.async_copy(lens.at[pl.ds(_al(jc * B), B)],
                         lenbuf.at[pl.ds(t * B, B)], rsem)
    for t in range(BLOCKS_PER_SUB):
        jc = jnp.minimum(s + NS * t, NBLOCKS - 1)
        pltpu.make_async_copy(lens.at[pl.ds(_al(jc * B), B)],
                              lenbuf.at[pl.ds(t * B, B)], rsem).wait()
    for t in range(BLOCKS_PER_SUB):
        j = s + NS * t
        acc = jnp.zeros((LN,), jnp.int32)
        for q in range(B // LN):
            acc = acc + lenbuf[pl.ds(t * B + q * LN, LN)]
        for kk in (1, 2, 4, 8):
            acc = acc + jnp.take(acc, jnp.bitwise_xor(iota, kk))
        ssum = jnp.where(j < NBLOCKS, acc[0], 0)
        plsc.store_scatter(srow_v, [jnp.full((LN,), t, jnp.int32)],
                           jnp.full((LN,), ssum, jnp.int32))
    pltpu.sync_copy(srow_v, spm_sums.at[pl.ds(_al(s * LN), LN)])
    plsc.subcore_barrier()
    pltpu.sync_copy(spm_sums, sums_v)

    cin = jnp.int32(0)
    for v in range(NPAD // LN):
        j = v * LN + iota
        idx = (j % NS) * LN + (j // NS)
        bsv = plsc.load_gather(sums_v, [idx])
        bsv = jnp.where(j < NBLOCKS, bsv, 0)
        incl = plsc.cumsum(bsv)
        bs_v[pl.ds(v * LN, LN)] = bsv
        inoff_v[pl.ds(v * LN, LN)] = incl - bsv + cin
        cin = cin + jnp.sum(bsv)

    cout = jnp.int32(0)
    for v in range(NPAD // LN):
        i = v * LN + iota
        ic = jnp.minimum(i, NBLOCKS - 1)
        rc = plsc.load_gather(recat_v, [ic])
        pbs = plsc.load_gather(bs_v, [rc])
        pbs = jnp.where(i < NBLOCKS, pbs, 0)
        sb = plsc.load_gather(inoff_v, [rc])
        incl = plsc.cumsum(pbs)
        src_v[pl.ds(v * LN, LN)] = sb
        dst_v[pl.ds(v * LN, LN)] = incl - pbs + cout
        cout = cout + jnp.sum(pbs)

    rowsrc = [None] * BLOCKS_PER_WORKER
    for t in range(BLOCKS_PER_WORKER):
        i = wid + NW * t
        ic = jnp.minimum(i, NBLOCKS - 1)
        rowsrc[t] = _scalar_at(recat_v, ic)

        @pl.when(i < NBLOCKS)
        def _rd(r=rowsrc[t], t=t):
            pltpu.async_copy(lens.at[pl.ds(_al(r * B), B)],
                             lenbuf.at[pl.ds(t * B, B)], rsem)
    for t in range(BLOCKS_PER_WORKER):
        i = wid + NW * t

        @pl.when(i < NBLOCKS)
        def _rdw(r=rowsrc[t], t=t):
            pltpu.make_async_copy(lens.at[pl.ds(_al(r * B), B)],
                                  lenbuf.at[pl.ds(t * B, B)], rsem).wait()
    for t in range(BLOCKS_PER_WORKER):
        i = wid + NW * t
        ic = jnp.minimum(i, NBLOCKS - 1)

        @pl.when(i < NBLOCKS)
        def _wr(ic=ic, t=t):
            pltpu.async_copy(lenbuf.at[pl.ds(t * B, B)],
                             lout.at[pl.ds(_al(ic * B), B)], lsem)

    def assemble(g, asmb):
        o0 = _al(g * C)
        acc_le = jnp.zeros((LN,), jnp.int32)
        acc_lt = jnp.zeros((LN,), jnp.int32)
        for v in range(NPAD // LN):
            dv = dst_v[pl.ds(v * LN, LN)]
            acc_le = acc_le + (dv <= o0).astype(jnp.int32)
            acc_lt = acc_lt + (dv < o0 + C).astype(jnp.int32)
        for kk in (1, 2, 4, 8):
            acc_le = acc_le + jnp.take(acc_le, jnp.bitwise_xor(iota, kk))
            acc_lt = acc_lt + jnp.take(acc_lt, jnp.bitwise_xor(iota, kk))
        jlo = acc_le[0] - 1
        jhi = acc_lt[0]

        def seg_body(j, _):
            dj = _scalar_at(dst_v, j)
            dj1 = _scalar_at(dst_v, j + 1)
            a = jnp.maximum(dj - o0, 0)
            b = jnp.minimum(dj1 - o0, C)

            @pl.when(b > a)
            def _seg(j=j, a=a, b=b, dj=dj):
                sj = _scalar_at(src_v, j)
                sA = sj + (o0 + a - dj)
                sh = jnp.bitwise_and(sA, 7)
                sA8 = sA - sh
                s0 = jnp.minimum(sA8, TOTAL - R)
                e = (sA8 - s0) + sh + (b - a)
                nr = (e + R - 1) // R

                def fire(q, _):
                    so = jnp.minimum(s0 + q * R, TOTAL - R)
                    pltpu.async_copy(
                        vals.at[pl.ds(_al(so), R)],
                        stage.at[pl.ds(_al(so - s0), R)], rsem)
                    return 0

                def drain(q, _):
                    so = jnp.minimum(s0 + q * R, TOTAL - R)
                    pltpu.make_async_copy(
                        vals.at[pl.ds(_al(so), R)],
                        stage.at[pl.ds(_al(so - s0), R)], rsem).wait()
                    return 0

                lax.fori_loop(0, nr, fire, 0)
                lax.fori_loop(0, nr, drain, 0)

                d = (sA - s0) - a
                va = a // LN
                vb = (b - 1) // LN

                def edge(v):
                    base = _al(v * LN, LN)
                    x = base + iota
                    m = (x >= a) & (x < b)
                    gi = jnp.clip(x + d, 0, STAGE - 1)
                    gv = plsc.load_gather(stage, [gi])
                    old = asmb[pl.ds(base, LN)]
                    asmb[pl.ds(base, LN)] = jnp.where(m, gv, old)

                edge(va)

                @pl.when(vb > va)
                def _hi():
                    edge(vb)

                @plsc.parallel_loop(va + 1, vb, unroll=16)
                def _interior(v):
                    base = _al(v * LN, LN)
                    gv = plsc.load_gather(stage, [base + iota + d])
                    asmb[pl.ds(base, LN)] = gv

            return 0

        lax.fori_loop(jlo, jhi, seg_body, 0)

    def chunk_pair(tt, _):
        g0 = wid + NW * (2 * tt)
        g1 = g0 + NW

        @pl.when(g0 < NCHUNK)
        def _c0(g0=g0):
            assemble(g0, asm0)
            pltpu.async_copy(asm0, vout.at[pl.ds(_al(g0 * C), C)], wsem)

        @pl.when(g1 < NCHUNK)
        def _c1(g1=g1):
            assemble(g1, asm1)
            pltpu.async_copy(asm1, vout.at[pl.ds(_al(g1 * C), C)], wsem)

        @pl.when(g0 < NCHUNK)
        def _w0(g0=g0):
            pltpu.make_async_copy(asm0, vout.at[pl.ds(_al(g0 * C), C)],
                                  wsem).wait()

        @pl.when(g1 < NCHUNK)
        def _w1(g1=g1):
            pltpu.make_async_copy(asm1, vout.at[pl.ds(_al(g1 * C), C)],
                                  wsem).wait()

        return 0

    lax.fori_loop(0, NPAIR, chunk_pair, 0)

    for t in range(BLOCKS_PER_WORKER):
        i = wid + NW * t
        ic = jnp.minimum(i, NBLOCKS - 1)

        @pl.when(i < NBLOCKS)
        def _wrw(ic=ic, t=t):
            pltpu.make_async_copy(lenbuf.at[pl.ds(t * B, B)],
                                  lout.at[pl.ds(_al(ic * B), B)], lsem).wait()


@functools.partial(
    pl.kernel,
    out_type=[
        jax.ShapeDtypeStruct((TOTAL,), jnp.float32),
        jax.ShapeDtypeStruct((NBLOCKS * B,), jnp.int32),
    ],
    mesh=plsc.VectorSubcoreMesh(core_axis_name="c", subcore_axis_name="s"),
    scratch_types=[
        pltpu.VMEM((NPAD,), jnp.int32),
        pltpu.VMEM((BLOCKS_PER_SUB * B,), jnp.int32),
        pltpu.VMEM((LN,), jnp.int32),
        pltpu.VMEM((NS * LN,), jnp.int32),
        pltpu.VMEM((NPAD,), jnp.int32),
        pltpu.VMEM((NPAD,), jnp.int32),
        pltpu.VMEM((NPAD,), jnp.int32),
        pltpu.VMEM((NPAD,), jnp.int32),
        pltpu.VMEM((C,), jnp.float32),
        pltpu.VMEM((C,), jnp.float32),
        pltpu.VMEM((STAGE,), jnp.float32),
        pltpu.VMEM_SHARED((NS * LN,), jnp.int32),
        pltpu.SemaphoreType.DMA,
        pltpu.SemaphoreType.DMA,
        pltpu.SemaphoreType.DMA,
    ],
    compiler_params=pltpu.CompilerParams(needs_layout_passes=False),
)
def _kjt_recat(vals, lens, recat_h, vout, lout, *scratch):
    _body(vals, lens, recat_h, vout, lout, *scratch)


def kernel(values, lengths):
    recat = jnp.asarray(np.pad(_recat_perm(), (0, NPAD - NBLOCKS)))
    values_out, lengths_out = _kjt_recat(values, lengths, recat)
    return values_out, lengths_out


# speedup vs baseline: 5165.7914x; 1.1164x over previous
import functools

import numpy as np
import jax
import jax.numpy as jnp
from jax import lax
from jax.experimental import pallas as pl
from jax.experimental.pallas import tpu as pltpu
from jax.experimental.pallas import tpu_sc as plsc

W = 8
LOCAL_SPLIT = 13
B = 1024
AVG_LEN = 20
NBLOCKS = W * LOCAL_SPLIT
TOTAL = NBLOCKS * B * AVG_LEN
NPAD = 112

_info = plsc.get_sparse_core_info()
NC, NS, LN = _info.num_cores, _info.num_subcores, _info.num_lanes
NW = NC * NS
BLOCKS_PER_SUB = -(-NBLOCKS // NS)
BLOCKS_PER_WORKER = -(-NBLOCKS // NW)

C = 8192
NCHUNK = TOTAL // C
CH_PER_W = -(-NCHUNK // NW)
NPAIR = (CH_PER_W + 1) // 2
R = 2048
NSTG = (C + 7 + R - 1) // R
STAGE = NSTG * R


def _recat_perm() -> np.ndarray:
    out = []
    for i in range(LOCAL_SPLIT):
        for j in range(W):
            out.append(i + j * LOCAL_SPLIT)
    return np.array(out, dtype=np.int32)


def _scalar_at(ref, i):
    return plsc.load_gather(ref, [jnp.full((LN,), i, jnp.int32)])[0]


def _al(x, n=8):
    return pl.multiple_of(x, n)


def _body(vals, lens, recat_h, vout, lout,
          recat_v, lenbuf, srow_v, sums_v,
          bs_v, inoff_v, src_v, dst_v, asm0, asm1, stage, spm_sums,
          rsem, psem, osem, wsem, lsem):
    c = lax.axis_index("c")
    s = lax.axis_index("s")
    wid = s * NC + c
    iota = lax.iota(jnp.int32, LN)

    pltpu.sync_copy(recat_h, recat_v)

    srow_v[...] = jnp.zeros((LN,), jnp.int32)
    for t in range(BLOCKS_PER_SUB):
        jc = jnp.minimum(s + NS * t, NBLOCKS - 1)
        pltpu.async_copy(lens.at[pl.ds(_al(jc * B), B)],
                         lenbuf.at[pl.ds(t * B, B)], rsem)
    for t in range(BLOCKS_PER_SUB):
        jc = jnp.minimum(s + NS * t, NBLOCKS - 1)
        pltpu.make_async_copy(lens.at[pl.ds(_al(jc * B), B)],
                              lenbuf.at[pl.ds(t * B, B)], rsem).wait()
    for t in range(BLOCKS_PER_SUB):
        j = s + NS * t
        acc = jnp.zeros((LN,), jnp.int32)
        for q in range(B // LN):
            acc = acc + lenbuf[pl.ds(t * B + q * LN, LN)]
        for kk in (1, 2, 4, 8):
            acc = acc + jnp.take(acc, jnp.bitwise_xor(iota, kk))
        ssum = jnp.where(j < NBLOCKS, acc[0], 0)
        plsc.store_scatter(srow_v, [jnp.full((LN,), t, jnp.int32)],
                           jnp.full((LN,), ssum, jnp.int32))
    pltpu.sync_copy(srow_v, spm_sums.at[pl.ds(_al(s * LN), LN)])
    plsc.subcore_barrier()
    pltpu.sync_copy(spm_sums, sums_v)

    cin = jnp.int32(0)
    for v in range(NPAD // LN):
        j = v * LN + iota
        idx = (j % NS) * LN + (j // NS)
        bsv = plsc.load_gather(sums_v, [idx])
        bsv = jnp.where(j < NBLOCKS, bsv, 0)
        incl = plsc.cumsum(bsv)
        bs_v[pl.ds(v * LN, LN)] = bsv
        inoff_v[pl.ds(v * LN, LN)] = incl - bsv + cin
        cin = cin + jnp.sum(bsv)

    cout = jnp.int32(0)
    for v in range(NPAD // LN):
        i = v * LN + iota
        ic = jnp.minimum(i, NBLOCKS - 1)
        rc = plsc.load_gather(recat_v, [ic])
        pbs = plsc.load_gather(bs_v, [rc])
        pbs = jnp.where(i < NBLOCKS, pbs, 0)
        sb = plsc.load_gather(inoff_v, [rc])
        incl = plsc.cumsum(pbs)
        src_v[pl.ds(v * LN, LN)] = sb
        dst_v[pl.ds(v * LN, LN)] = incl - pbs + cout
        cout = cout + jnp.sum(pbs)

    rowsrc = [None] * BLOCKS_PER_WORKER
    for t in range(BLOCKS_PER_WORKER):
        i = wid + NW * t
        ic = jnp.minimum(i, NBLOCKS - 1)
        rowsrc[t] = _scalar_at(recat_v, ic)

        @pl.when(i < NBLOCKS)
        def _rd(r=rowsrc[t], t=t):
            pltpu.async_copy(lens.at[pl.ds(_al(r * B), B)],
                             lenbuf.at[pl.ds(t * B, B)], rsem)
    for t in range(BLOCKS_PER_WORKER):
        i = wid + NW * t

        @pl.when(i < NBLOCKS)
        def _rdw(r=rowsrc[t], t=t):
            pltpu.make_async_copy(lens.at[pl.ds(_al(r * B), B)],
                                  lenbuf.at[pl.ds(t * B, B)], rsem).wait()
    for t in range(BLOCKS_PER_WORKER):
        i = wid + NW * t
        ic = jnp.minimum(i, NBLOCKS - 1)

        @pl.when(i < NBLOCKS)
        def _wr(ic=ic, t=t):
            pltpu.async_copy(lenbuf.at[pl.ds(t * B, B)],
                             lout.at[pl.ds(_al(ic * B), B)], lsem)

    def seg_range(g):
        o0 = _al(g * C)
        acc_le = jnp.zeros((LN,), jnp.int32)
        acc_lt = jnp.zeros((LN,), jnp.int32)
        for v in range(NPAD // LN):
            dv = dst_v[pl.ds(v * LN, LN)]
            acc_le = acc_le + (dv <= o0).astype(jnp.int32)
            acc_lt = acc_lt + (dv < o0 + C).astype(jnp.int32)
        for kk in (1, 2, 4, 8):
            acc_le = acc_le + jnp.take(acc_le, jnp.bitwise_xor(iota, kk))
            acc_lt = acc_lt + jnp.take(acc_lt, jnp.bitwise_xor(iota, kk))
        return acc_le[0] - 1, acc_lt[0]

    def seg_geom(g, j):
        o0 = _al(g * C)
        dj = _scalar_at(dst_v, j)
        dj1 = _scalar_at(dst_v, j + 1)
        a = jnp.maximum(dj - o0, 0)
        b = jnp.minimum(dj1 - o0, C)
        sj = _scalar_at(src_v, j)
        sA = sj + (o0 + a - dj)
        sA8 = sA - jnp.bitwise_and(sA, 7)
        s0 = jnp.minimum(sA8, TOTAL - R)
        e = (sA - s0) + (b - a)
        return a, b, sA, s0, e

    def fire_seg(s0, e, soff, sem):
        def fire(q, _):
            so = jnp.minimum(s0 + q * R, TOTAL - R)
            pltpu.async_copy(vals.at[pl.ds(_al(so), R)],
                             stage.at[pl.ds(_al(soff + so - s0), R)], sem)
            return 0
        lax.fori_loop(0, (e + R - 1) // R, fire, 0)

    def drain_seg(s0, e, soff, sem):
        def drain(q, _):
            so = jnp.minimum(s0 + q * R, TOTAL - R)
            pltpu.make_async_copy(vals.at[pl.ds(_al(so), R)],
                                  stage.at[pl.ds(_al(soff + so - s0), R)],
                                  sem).wait()
            return 0
        lax.fori_loop(0, (e + R - 1) // R, drain, 0)

    def gather_seg(a, b, sA, s0, soff, asmb):
        d = soff + (sA - s0) - a
        va = a // LN
        vb = (b - 1) // LN

        def edge(v):
            base = _al(v * LN, LN)
            x = base + iota
            m = (x >= a) & (x < b)
            gi = jnp.clip(x + d, 0, 3 * STAGE - 1)
            gv = plsc.load_gather(stage, [gi])
            old = asmb[pl.ds(base, LN)]
            asmb[pl.ds(base, LN)] = jnp.where(m, gv, old)

        edge(va)

        @pl.when(vb > va)
        def _hi():
            edge(vb)

        @plsc.parallel_loop(va + 1, vb, unroll=16)
        def _interior(v):
            base = _al(v * LN, LN)
            gv = plsc.load_gather(stage, [base + iota + d])
            asmb[pl.ds(base, LN)] = gv

    def fire_first(g, soff, sem):
        jlo, _ = seg_range(g)
        _, _, _, s0, e = seg_geom(g, jlo)
        fire_seg(s0, e, soff, sem)

    def gather_chunk(g, soff, sem, asmb):
        jlo, jhi = seg_range(g)
        a0, b0, sA0, s00, e0 = seg_geom(g, jlo)
        drain_seg(s00, e0, soff, sem)
        gather_seg(a0, b0, sA0, s00, soff, asmb)

        def seg_body(j, _):
            a, b, sA, s0, e = seg_geom(g, j)

            @pl.when(b > a)
            def _seg(a=a, b=b, sA=sA, s0=s0, e=e):
                fire_seg(s0, e, 2 * STAGE, osem)
                drain_seg(s0, e, 2 * STAGE, osem)
                gather_seg(a, b, sA, s0, 2 * STAGE, asmb)

            return 0

        lax.fori_loop(jlo + 1, jhi, seg_body, 0)
        pltpu.async_copy(asmb, vout.at[pl.ds(_al(g * C), C)], wsem)

    gfirst = wid

    @pl.when(gfirst < NCHUNK)
    def _pro():
        fire_first(gfirst, 0, rsem)

    def chunk_pair(tt, _):
        g0 = wid + NW * (2 * tt)
        g1 = g0 + NW
        g2 = g1 + NW

        @pl.when(g1 < NCHUNK)
        def _p1(g1=g1):
            fire_first(g1, STAGE, psem)

        @pl.when(g0 < NCHUNK)
        def _c0(g0=g0):
            gather_chunk(g0, 0, rsem, asm0)

        @pl.when(g2 < NCHUNK)
        def _p2(g2=g2):
            fire_first(g2, 0, rsem)

        @pl.when(g1 < NCHUNK)
        def _c1(g1=g1):
            gather_chunk(g1, STAGE, psem, asm1)

        @pl.when(g0 < NCHUNK)
        def _w0(g0=g0):
            pltpu.make_async_copy(asm0, vout.at[pl.ds(_al(g0 * C), C)],
                                  wsem).wait()

        @pl.when(g1 < NCHUNK)
        def _w1(g1=g1):
            pltpu.make_async_copy(asm1, vout.at[pl.ds(_al(g1 * C), C)],
                                  wsem).wait()

        return 0

    lax.fori_loop(0, NPAIR, chunk_pair, 0)

    for t in range(BLOCKS_PER_WORKER):
        i = wid + NW * t
        ic = jnp.minimum(i, NBLOCKS - 1)

        @pl.when(i < NBLOCKS)
        def _wrw(ic=ic, t=t):
            pltpu.make_async_copy(lenbuf.at[pl.ds(t * B, B)],
                                  lout.at[pl.ds(_al(ic * B), B)], lsem).wait()


@functools.partial(
    pl.kernel,
    out_type=[
        jax.ShapeDtypeStruct((TOTAL,), jnp.float32),
        jax.ShapeDtypeStruct((NBLOCKS * B,), jnp.int32),
    ],
    mesh=plsc.VectorSubcoreMesh(core_axis_name="c", subcore_axis_name="s"),
    scratch_types=[
        pltpu.VMEM((NPAD,), jnp.int32),
        pltpu.VMEM((BLOCKS_PER_SUB * B,), jnp.int32),
        pltpu.VMEM((LN,), jnp.int32),
        pltpu.VMEM((NS * LN,), jnp.int32),
        pltpu.VMEM((NPAD,), jnp.int32),
        pltpu.VMEM((NPAD,), jnp.int32),
        pltpu.VMEM((NPAD,), jnp.int32),
        pltpu.VMEM((NPAD,), jnp.int32),
        pltpu.VMEM((C,), jnp.float32),
        pltpu.VMEM((C,), jnp.float32),
        pltpu.VMEM((3 * STAGE,), jnp.float32),
        pltpu.VMEM_SHARED((NS * LN,), jnp.int32),
        pltpu.SemaphoreType.DMA,
        pltpu.SemaphoreType.DMA,
        pltpu.SemaphoreType.DMA,
        pltpu.SemaphoreType.DMA,
        pltpu.SemaphoreType.DMA,
    ],
    compiler_params=pltpu.CompilerParams(needs_layout_passes=False),
)
def _kjt_recat(vals, lens, recat_h, vout, lout, *scratch):
    _body(vals, lens, recat_h, vout, lout, *scratch)


def kernel(values, lengths):
    recat = jnp.asarray(np.pad(_recat_perm(), (0, NPAD - NBLOCKS)))
    values_out, lengths_out = _kjt_recat(values, lengths, recat)
    return values_out, lengths_out


# speedup vs baseline: 5276.1467x; 1.0214x over previous
import functools

import numpy as np
import jax
import jax.numpy as jnp
from jax import lax
from jax.experimental import pallas as pl
from jax.experimental.pallas import tpu as pltpu
from jax.experimental.pallas import tpu_sc as plsc

W = 8
LOCAL_SPLIT = 13
B = 1024
AVG_LEN = 20
NBLOCKS = W * LOCAL_SPLIT
TOTAL = NBLOCKS * B * AVG_LEN
NPAD = 112

_info = plsc.get_sparse_core_info()
NC, NS, LN = _info.num_cores, _info.num_subcores, _info.num_lanes
NW = NC * NS
BLOCKS_PER_SUB = -(-NBLOCKS // NS)
BLOCKS_PER_WORKER = -(-NBLOCKS // NW)

C = 16384
NCHUNK = TOTAL // C
CH_PER_W = -(-NCHUNK // NW)
NPAIR = (CH_PER_W + 1) // 2
R = 2048
NSTG = (C + 7 + R - 1) // R
STAGE = NSTG * R


def _recat_perm() -> np.ndarray:
    out = []
    for i in range(LOCAL_SPLIT):
        for j in range(W):
            out.append(i + j * LOCAL_SPLIT)
    return np.array(out, dtype=np.int32)


def _scalar_at(ref, i):
    return plsc.load_gather(ref, [jnp.full((LN,), i, jnp.int32)])[0]


def _al(x, n=8):
    return pl.multiple_of(x, n)


def _body(vals, lens, recat_h, vout, lout,
          recat_v, lenbuf, srow_v, sums_v,
          bs_v, inoff_v, src_v, dst_v, asm0, asm1, stage, spm_sums,
          rsem, psem, osem, wsem, lsem):
    c = lax.axis_index("c")
    s = lax.axis_index("s")
    wid = s * NC + c
    iota = lax.iota(jnp.int32, LN)

    pltpu.sync_copy(recat_h, recat_v)

    srow_v[...] = jnp.zeros((LN,), jnp.int32)
    for t in range(BLOCKS_PER_SUB):
        jc = jnp.minimum(s + NS * t, NBLOCKS - 1)
        pltpu.async_copy(lens.at[pl.ds(_al(jc * B), B)],
                         lenbuf.at[pl.ds(t * B, B)], rsem)
    for t in range(BLOCKS_PER_SUB):
        jc = jnp.minimum(s + NS * t, NBLOCKS - 1)
        pltpu.make_async_copy(lens.at[pl.ds(_al(jc * B), B)],
                              lenbuf.at[pl.ds(t * B, B)], rsem).wait()
    for t in range(BLOCKS_PER_SUB):
        j = s + NS * t
        acc = jnp.zeros((LN,), jnp.int32)
        for q in range(B // LN):
            acc = acc + lenbuf[pl.ds(t * B + q * LN, LN)]
        for kk in (1, 2, 4, 8):
            acc = acc + jnp.take(acc, jnp.bitwise_xor(iota, kk))
        ssum = jnp.where(j < NBLOCKS, acc[0], 0)
        plsc.store_scatter(srow_v, [jnp.full((LN,), t, jnp.int32)],
                           jnp.full((LN,), ssum, jnp.int32))
    pltpu.sync_copy(srow_v, spm_sums.at[pl.ds(_al(s * LN), LN)])
    plsc.subcore_barrier()
    pltpu.sync_copy(spm_sums, sums_v)

    cin = jnp.int32(0)
    for v in range(NPAD // LN):
        j = v * LN + iota
        idx = (j % NS) * LN + (j // NS)
        bsv = plsc.load_gather(sums_v, [idx])
        bsv = jnp.where(j < NBLOCKS, bsv, 0)
        incl = plsc.cumsum(bsv)
        bs_v[pl.ds(v * LN, LN)] = bsv
        inoff_v[pl.ds(v * LN, LN)] = incl - bsv + cin
        cin = cin + jnp.sum(bsv)

    cout = jnp.int32(0)
    for v in range(NPAD // LN):
        i = v * LN + iota
        ic = jnp.minimum(i, NBLOCKS - 1)
        rc = plsc.load_gather(recat_v, [ic])
        pbs = plsc.load_gather(bs_v, [rc])
        pbs = jnp.where(i < NBLOCKS, pbs, 0)
        sb = plsc.load_gather(inoff_v, [rc])
        incl = plsc.cumsum(pbs)
        src_v[pl.ds(v * LN, LN)] = sb
        dst_v[pl.ds(v * LN, LN)] = incl - pbs + cout
        cout = cout + jnp.sum(pbs)

    rowsrc = [None] * BLOCKS_PER_WORKER
    for t in range(BLOCKS_PER_WORKER):
        i = wid + NW * t
        ic = jnp.minimum(i, NBLOCKS - 1)
        rowsrc[t] = _scalar_at(recat_v, ic)

        @pl.when(i < NBLOCKS)
        def _rd(r=rowsrc[t], t=t):
            pltpu.async_copy(lens.at[pl.ds(_al(r * B), B)],
                             lenbuf.at[pl.ds(t * B, B)], rsem)
    for t in range(BLOCKS_PER_WORKER):
        i = wid + NW * t

        @pl.when(i < NBLOCKS)
        def _rdw(r=rowsrc[t], t=t):
            pltpu.make_async_copy(lens.at[pl.ds(_al(r * B), B)],
                                  lenbuf.at[pl.ds(t * B, B)], rsem).wait()
    for t in range(BLOCKS_PER_WORKER):
        i = wid + NW * t
        ic = jnp.minimum(i, NBLOCKS - 1)

        @pl.when(i < NBLOCKS)
        def _wr(ic=ic, t=t):
            pltpu.async_copy(lenbuf.at[pl.ds(t * B, B)],
                             lout.at[pl.ds(_al(ic * B), B)], lsem)

    def seg_range(g):
        o0 = _al(g * C)
        acc_le = jnp.zeros((LN,), jnp.int32)
        acc_lt = jnp.zeros((LN,), jnp.int32)
        for v in range(NPAD // LN):
            dv = dst_v[pl.ds(v * LN, LN)]
            acc_le = acc_le + (dv <= o0).astype(jnp.int32)
            acc_lt = acc_lt + (dv < o0 + C).astype(jnp.int32)
        for kk in (1, 2, 4, 8):
            acc_le = acc_le + jnp.take(acc_le, jnp.bitwise_xor(iota, kk))
            acc_lt = acc_lt + jnp.take(acc_lt, jnp.bitwise_xor(iota, kk))
        return acc_le[0] - 1, acc_lt[0]

    def seg_geom(g, j):
        o0 = _al(g * C)
        dj = _scalar_at(dst_v, j)
        dj1 = _scalar_at(dst_v, j + 1)
        a = jnp.maximum(dj - o0, 0)
        b = jnp.minimum(dj1 - o0, C)
        sj = _scalar_at(src_v, j)
        sA = sj + (o0 + a - dj)
        sA8 = sA - jnp.bitwise_and(sA, 7)
        s0 = jnp.minimum(sA8, TOTAL - R)
        e = (sA - s0) + (b - a)
        return a, b, sA, s0, e

    def fire_seg(s0, e, soff, sem):
        def fire(q, _):
            so = jnp.minimum(s0 + q * R, TOTAL - R)
            pltpu.async_copy(vals.at[pl.ds(_al(so), R)],
                             stage.at[pl.ds(_al(soff + so - s0), R)], sem)
            return 0
        lax.fori_loop(0, (e + R - 1) // R, fire, 0)

    def drain_seg(s0, e, soff, sem):
        def drain(q, _):
            so = jnp.minimum(s0 + q * R, TOTAL - R)
            pltpu.make_async_copy(vals.at[pl.ds(_al(so), R)],
                                  stage.at[pl.ds(_al(soff + so - s0), R)],
                                  sem).wait()
            return 0
        lax.fori_loop(0, (e + R - 1) // R, drain, 0)

    def gather_seg(a, b, sA, s0, soff, asmb):
        d = soff + (sA - s0) - a
        va = a // LN
        vb = (b - 1) // LN

        def edge(v):
            base = _al(v * LN, LN)
            x = base + iota
            m = (x >= a) & (x < b)
            gi = jnp.clip(x + d, 0, 3 * STAGE - 1)
            gv = plsc.load_gather(stage, [gi])
            old = asmb[pl.ds(base, LN)]
            asmb[pl.ds(base, LN)] = jnp.where(m, gv, old)

        edge(va)

        @pl.when(vb > va)
        def _hi():
            edge(vb)

        @plsc.parallel_loop(va + 1, vb, unroll=16)
        def _interior(v):
            base = _al(v * LN, LN)
            gv = plsc.load_gather(stage, [base + iota + d])
            asmb[pl.ds(base, LN)] = gv

    def fire_first(g, soff, sem):
        jlo, _ = seg_range(g)
        _, _, _, s0, e = seg_geom(g, jlo)
        fire_seg(s0, e, soff, sem)

    def gather_chunk(g, soff, sem, asmb):
        jlo, jhi = seg_range(g)
        a0, b0, sA0, s00, e0 = seg_geom(g, jlo)
        drain_seg(s00, e0, soff, sem)
        gather_seg(a0, b0, sA0, s00, soff, asmb)

        def seg_body(j, _):
            a, b, sA, s0, e = seg_geom(g, j)

            @pl.when(b > a)
            def _seg(a=a, b=b, sA=sA, s0=s0, e=e):
                fire_seg(s0, e, 2 * STAGE, osem)
                drain_seg(s0, e, 2 * STAGE, osem)
                gather_seg(a, b, sA, s0, 2 * STAGE, asmb)

            return 0

        lax.fori_loop(jlo + 1, jhi, seg_body, 0)
        pltpu.async_copy(asmb, vout.at[pl.ds(_al(g * C), C)], wsem)

    gfirst = wid

    @pl.when(gfirst < NCHUNK)
    def _pro():
        fire_first(gfirst, 0, rsem)

    def chunk_pair(tt, _):
        g0 = wid + NW * (2 * tt)
        g1 = g0 + NW
        g2 = g1 + NW

        @pl.when(g1 < NCHUNK)
        def _p1(g1=g1):
            fire_first(g1, STAGE, psem)

        @pl.when(g0 < NCHUNK)
        def _c0(g0=g0):
            gather_chunk(g0, 0, rsem, asm0)

        @pl.when(g2 < NCHUNK)
        def _p2(g2=g2):
            fire_first(g2, 0, rsem)

        @pl.when(g1 < NCHUNK)
        def _c1(g1=g1):
            gather_chunk(g1, STAGE, psem, asm1)

        @pl.when(g0 < NCHUNK)
        def _w0(g0=g0):
            pltpu.make_async_copy(asm0, vout.at[pl.ds(_al(g0 * C), C)],
                                  wsem).wait()

        @pl.when(g1 < NCHUNK)
        def _w1(g1=g1):
            pltpu.make_async_copy(asm1, vout.at[pl.ds(_al(g1 * C), C)],
                                  wsem).wait()

        return 0

    lax.fori_loop(0, NPAIR, chunk_pair, 0)

    for t in range(BLOCKS_PER_WORKER):
        i = wid + NW * t
        ic = jnp.minimum(i, NBLOCKS - 1)

        @pl.when(i < NBLOCKS)
        def _wrw(ic=ic, t=t):
            pltpu.make_async_copy(lenbuf.at[pl.ds(t * B, B)],
                                  lout.at[pl.ds(_al(ic * B), B)], lsem).wait()


@functools.partial(
    pl.kernel,
    out_type=[
        jax.ShapeDtypeStruct((TOTAL,), jnp.float32),
        jax.ShapeDtypeStruct((NBLOCKS * B,), jnp.int32),
    ],
    mesh=plsc.VectorSubcoreMesh(core_axis_name="c", subcore_axis_name="s"),
    scratch_types=[
        pltpu.VMEM((NPAD,), jnp.int32),
        pltpu.VMEM((BLOCKS_PER_SUB * B,), jnp.int32),
        pltpu.VMEM((LN,), jnp.int32),
        pltpu.VMEM((NS * LN,), jnp.int32),
        pltpu.VMEM((NPAD,), jnp.int32),
        pltpu.VMEM((NPAD,), jnp.int32),
        pltpu.VMEM((NPAD,), jnp.int32),
        pltpu.VMEM((NPAD,), jnp.int32),
        pltpu.VMEM((C,), jnp.float32),
        pltpu.VMEM((C,), jnp.float32),
        pltpu.VMEM((3 * STAGE,), jnp.float32),
        pltpu.VMEM_SHARED((NS * LN,), jnp.int32),
        pltpu.SemaphoreType.DMA,
        pltpu.SemaphoreType.DMA,
        pltpu.SemaphoreType.DMA,
        pltpu.SemaphoreType.DMA,
        pltpu.SemaphoreType.DMA,
    ],
    compiler_params=pltpu.CompilerParams(needs_layout_passes=False),
)
def _kjt_recat(vals, lens, recat_h, vout, lout, *scratch):
    _body(vals, lens, recat_h, vout, lout, *scratch)


def kernel(values, lengths):
    recat = jnp.asarray(np.pad(_recat_perm(), (0, NPAD - NBLOCKS)))
    values_out, lengths_out = _kjt_recat(values, lengths, recat)
    return values_out, lengths_out
